# Initial kernel scaffold; baseline (speedup 1.0000x reference)
import jax
import jax.numpy as jnp
from jax.experimental import pallas as pl


def kernel(x, edge_index, W1, b1, W2, b2, W3, b3):
    raise NotImplementedError("write your pallas kernel here")


# speedup vs baseline: 8.5474x; 8.5474x over previous
import functools

import jax
import jax.numpy as jnp
from jax import lax
from jax.experimental import pallas as pl
from jax.experimental.pallas import tpu as pltpu
from jax.experimental.pallas import tpu_sc as plsc

N = 10000
N_PAD = 10240
E = 160000
E_PAD = 161792
DUMMY = 10016
RB = 256

_mesh = plsc.VectorSubcoreMesh(core_axis_name="c", subcore_axis_name="s")
_sc_params = pltpu.CompilerParams(needs_layout_passes=False)


@functools.partial(
    pl.kernel,
    out_type=jax.ShapeDtypeStruct((32, N_PAD), jnp.float32),
    mesh=_mesh,
    scratch_types=[
        pltpu.VMEM((316, 16), jnp.int32),
        pltpu.VMEM((N_PAD,), jnp.float32),
    ],
    compiler_params=_sc_params,
)
def _deg_kernel(dst_hbm, out_hbm, idx_v, deg_v):
    cid = lax.axis_index("c")
    sid = lax.axis_index("s")
    wid = sid * 2 + cid

    def zero_body(i, _):
        deg_v[pl.ds(i * 16, 16)] = jnp.zeros((16,), jnp.float32)
        return 0

    lax.fori_loop(0, N_PAD // 16, zero_body, 0)
    pltpu.sync_copy(dst_hbm.at[wid], idx_v)
    ones = jnp.ones((16,), jnp.float32)

    def body(j, _):
        plsc.addupdate_scatter(deg_v, [idx_v[j]], ones)
        return 0

    lax.fori_loop(0, 316, body, 0)
    pltpu.sync_copy(deg_v, out_hbm.at[wid])


def _make_prop(nchunks):
    rounds = nchunks // 2
    ept = E_PAD // 16
    nb = ept // 128

    @functools.partial(
        pl.kernel,
        out_type=jax.ShapeDtypeStruct((nchunks * N_PAD, 128), jnp.float32),
        mesh=_mesh,
        scratch_types=[
            pltpu.VMEM((nb, 128), jnp.int32),
            pltpu.VMEM((nb, 128), jnp.int32),
            pltpu.VMEM((128,), jnp.int32),
            pltpu.VMEM((128, 128), jnp.float32),
            pltpu.VMEM_SHARED((N_PAD, 128), jnp.float32),
            pltpu.SemaphoreType.DMA,
        ],
        compiler_params=_sc_params,
    )
    def prop(g_hbm, src_hbm, dst_hbm, out_hbm, src_v, dst_v, gidx_v, buf, acc, gsem):
        cid = lax.axis_index("c")
        sid = lax.axis_index("s")
        pltpu.sync_copy(src_hbm.at[sid], src_v)
        pltpu.sync_copy(dst_hbm.at[sid], dst_v)

        for r in range(rounds):
            chunk = r * 2 + cid
            base = chunk * N_PAD

            def zb(i, _):
                buf[i // 8, pl.ds((i % 8) * 16, 16)] = jnp.zeros((16,), jnp.float32)
                return 0

            lax.fori_loop(0, 1024, zb, 0)

            def zc(k, _):
                pltpu.sync_copy(buf, acc.at[pl.ds(sid * 640 + k * 128, 128)])
                return 0

            lax.fori_loop(0, 5, zc, 0)
            plsc.subcore_barrier()

            def mb(j, _):
                def ab(i, _):
                    sl = pl.ds(i * 16, 16)
                    gidx_v[sl] = src_v[j, sl] + base
                    return 0

                lax.fori_loop(0, 8, ab, 0)
                pltpu.async_copy(g_hbm.at[gidx_v], buf, gsem).wait()
                pltpu.sync_copy(buf, acc.at[dst_v.at[j]], add=True)
                return 0

            lax.fori_loop(0, nb, mb, 0)
            plsc.subcore_barrier()

            def ob(k, _):
                row0 = sid * 640 + k * 128
                pltpu.sync_copy(acc.at[pl.ds(row0, 128)],
                                out_hbm.at[pl.ds(base + row0, 128)])
                return 0

            lax.fori_loop(0, 5, ob, 0)
            if r + 1 < rounds:
                plsc.subcore_barrier()

    return prop


_prop2 = _make_prop(2)
_prop4 = _make_prop(4)


def _k1_body(parts_ref, x_ref, dinv_ref, g0_ref):
    deg = jnp.sum(parts_ref[...], axis=0) + 1.0
    dinv = lax.rsqrt(deg)
    dinv_ref[...] = dinv[:, None]
    g = x_ref[...] * dinv[:, None]
    g0_ref[0] = g[:, :128]
    g0_ref[1] = g[:, 128:]


def _k2_body(s_ref, g_ref, dinv_ref, w_ref, b_ref, out_ref):
    dinv = dinv_ref[...]
    p = jnp.concatenate([s_ref[0] + g_ref[0], s_ref[1] + g_ref[1]], axis=1) * dinv
    h = jnp.dot(p, w_ref[...], preferred_element_type=jnp.float32) + b_ref[...]
    h = jnp.maximum(h, 0.0) * dinv
    for c in range(4):
        out_ref[c] = h[:, c * 128:(c + 1) * 128]


def _k3_body(s_ref, g_ref, dinv_ref, w2_ref, b2_ref, w3_ref, out_ref):
    dinv = dinv_ref[...]
    p = jnp.concatenate([s_ref[c] + g_ref[c] for c in range(4)], axis=1) * dinv
    h2 = jnp.maximum(
        jnp.dot(p, w2_ref[...], preferred_element_type=jnp.float32) + b2_ref[...], 0.0)
    t = jnp.dot(h2, w3_ref[...], preferred_element_type=jnp.float32) * dinv
    out_ref[0] = t[:, :128]
    out_ref[1] = t[:, 128:]


def _k4_body(s_ref, g_ref, dinv_ref, b3_ref, out_ref):
    t = jnp.concatenate([s_ref[0] + g_ref[0], s_ref[1] + g_ref[1]], axis=1)
    out_ref[...] = t * dinv_ref[...] + b3_ref[...]


_k1 = pl.pallas_call(
    _k1_body,
    grid=(N_PAD // RB,),
    in_specs=[
        pl.BlockSpec((32, RB), lambda i: (0, i)),
        pl.BlockSpec((RB, 256), lambda i: (i, 0)),
    ],
    out_specs=[
        pl.BlockSpec((RB, 1), lambda i: (i, 0)),
        pl.BlockSpec((2, RB, 128), lambda i: (0, i, 0)),
    ],
    out_shape=[
        jax.ShapeDtypeStruct((N_PAD, 1), jnp.float32),
        jax.ShapeDtypeStruct((2, N_PAD, 128), jnp.float32),
    ],
)

_k2 = pl.pallas_call(
    _k2_body,
    grid=(N_PAD // RB,),
    in_specs=[
        pl.BlockSpec((2, RB, 128), lambda i: (0, i, 0)),
        pl.BlockSpec((2, RB, 128), lambda i: (0, i, 0)),
        pl.BlockSpec((RB, 1), lambda i: (i, 0)),
        pl.BlockSpec((256, 512), lambda i: (0, 0)),
        pl.BlockSpec((1, 512), lambda i: (0, 0)),
    ],
    out_specs=pl.BlockSpec((4, RB, 128), lambda i: (0, i, 0)),
    out_shape=jax.ShapeDtypeStruct((4, N_PAD, 128), jnp.float32),
)

_k3 = pl.pallas_call(
    _k3_body,
    grid=(N_PAD // RB,),
    in_specs=[
        pl.BlockSpec((4, RB, 128), lambda i: (0, i, 0)),
        pl.BlockSpec((4, RB, 128), lambda i: (0, i, 0)),
        pl.BlockSpec((RB, 1), lambda i: (i, 0)),
        pl.BlockSpec((512, 512), lambda i: (0, 0)),
        pl.BlockSpec((1, 512), lambda i: (0, 0)),
        pl.BlockSpec((512, 256), lambda i: (0, 0)),
    ],
    out_specs=pl.BlockSpec((2, RB, 128), lambda i: (0, i, 0)),
    out_shape=jax.ShapeDtypeStruct((2, N_PAD, 128), jnp.float32),
)

_k4 = pl.pallas_call(
    _k4_body,
    grid=(N_PAD // RB,),
    in_specs=[
        pl.BlockSpec((2, RB, 128), lambda i: (0, i, 0)),
        pl.BlockSpec((2, RB, 128), lambda i: (0, i, 0)),
        pl.BlockSpec((RB, 1), lambda i: (i, 0)),
        pl.BlockSpec((1, 256), lambda i: (0, 0)),
    ],
    out_specs=pl.BlockSpec((RB, 256), lambda i: (i, 0)),
    out_shape=jax.ShapeDtypeStruct((N_PAD, 256), jnp.float32),
)


def kernel(x, edge_index, W1, b1, W2, b2, W3, b3):
    ei = edge_index.astype(jnp.int32)
    npad = E_PAD - E
    src = jnp.concatenate([ei[0], jnp.zeros((npad,), jnp.int32)])
    dstp = jnp.concatenate([ei[1], jnp.full((npad,), DUMMY, jnp.int32)])
    dst_deg = dstp.reshape(32, 316, 16)
    src3 = src.reshape(16, 79, 128)
    dst3 = dstp.reshape(16, 79, 128)
    x_pad = jnp.pad(x, ((0, N_PAD - N), (0, 0)))

    parts = _deg_kernel(dst_deg)
    dinv, g0 = _k1(parts, x_pad)
    s0 = _prop2(g0.reshape(2 * N_PAD, 128), src3, dst3).reshape(2, N_PAD, 128)
    g1 = _k2(s0, g0, dinv, W1, b1.reshape(1, 512))
    s1 = _prop4(g1.reshape(4 * N_PAD, 128), src3, dst3).reshape(4, N_PAD, 128)
    g2 = _k3(s1, g1, dinv, W2, b2.reshape(1, 512), W3)
    s2 = _prop2(g2.reshape(2 * N_PAD, 128), src3, dst3).reshape(2, N_PAD, 128)
    out = _k4(s2, g2, dinv, b3.reshape(1, 256))
    return out[:N, :]
